# baseline (speedup 1.0000x reference)
import functools

import jax
import jax.numpy as jnp
from jax import lax
from jax.experimental import pallas as pl
from jax.experimental.pallas import tpu as pltpu
from jax.experimental.pallas import tpu_sc as plsc

D = 128
NC = 2
NS = 16
NW = NC * NS


def kernel(pos_enc_1D, pos):
    B, S = pos.shape
    V = pos_enc_1D.shape[0]
    N = B * S
    per_w = N // NW
    CH = 200
    n_ch = per_w // CH

    idx_flat = pos.reshape(N)
    mesh = plsc.VectorSubcoreMesh(core_axis_name="c", subcore_axis_name="s")

    @functools.partial(
        pl.kernel,
        mesh=mesh,
        out_type=jax.ShapeDtypeStruct((N, D), jnp.float32),
        scratch_types=[
            pltpu.VMEM((per_w,), jnp.int32),
            pltpu.VMEM((CH, D), jnp.float32),
            pltpu.VMEM((CH, D), jnp.float32),
            pltpu.VMEM((CH, D), jnp.float32),
            pltpu.VMEM((CH, D), jnp.float32),
            pltpu.VMEM_SHARED((V, D), jnp.float32),
            pltpu.SemaphoreType.DMA,
            pltpu.SemaphoreType.DMA,
            pltpu.SemaphoreType.DMA,
            pltpu.SemaphoreType.DMA,
            pltpu.SemaphoreType.DMA,
        ],
    )
    def gather_kernel(table_hbm, idx_hbm, out_hbm, idx_v, rows0, rows1,
                      rows2, rows3, table_sp, sem_i, sem_g0, sem_g1, sem_g2,
                      sem_g3):
        cid = lax.axis_index("c")
        sid = lax.axis_index("s")
        wid = sid * NC + cid
        base = wid * per_w

        idx_cp = pltpu.make_async_copy(
            idx_hbm.at[pl.ds(base, per_w)], idx_v, sem_i)
        idx_cp.start()

        @pl.when(sid == 0)
        def _():
            pltpu.sync_copy(table_hbm, table_sp)

        plsc.subcore_barrier()
        idx_cp.wait()

        def start_gather(i, rows, sem):
            pltpu.make_async_copy(
                table_sp.at[idx_v.at[pl.ds(i * CH, CH)]], rows, sem).start()

        def wait_gather(rows, sem):
            pltpu.make_async_copy(
                table_sp.at[idx_v.at[pl.ds(0, CH)]], rows, sem).wait()

        NB = 4
        bufs = ((rows0, sem_g0), (rows1, sem_g1), (rows2, sem_g2),
                (rows3, sem_g3))
        for k in range(NB - 1):
            start_gather(k, *bufs[k])

        def body(j, carry):
            i0 = NB * j
            for k in range(NB):
                rows, sem = bufs[k]
                nxt = i0 + k + NB - 1

                @pl.when(nxt < n_ch)
                def _():
                    nrows, nsem = bufs[(k + NB - 1) % NB]
                    start_gather(nxt, nrows, nsem)

                wait_gather(rows, sem)
                pltpu.sync_copy(
                    rows, out_hbm.at[pl.ds(base + (i0 + k) * CH, CH)])
            return carry

        lax.fori_loop(0, n_ch // NB, body, 0)

    out = gather_kernel(pos_enc_1D, idx_flat)
    return out.reshape(B, S, D)
